# baseline (speedup 1.0000x reference)
import dataclasses
import functools

import numpy as np
import jax
import jax.numpy as jnp
from jax import lax
from jax.experimental import pallas as pl
from jax.experimental.pallas import tpu as pltpu
from jax.experimental.pallas import tpu_sc as plsc

N = 16384
B = 4
P = 1024
NQ = B * P
S = 32
CF = 64
CO = 3 + CF
CP = 128
CO2 = 72
R2 = 0.2 * 0.2

NWORDS = N // 32


QB = 512
PCM = 4096
WPC = PCM // 32


def _pack_weights():
    w = np.zeros((PCM, 2 * WPC), np.float32)
    p = np.arange(PCM)
    g = p // 32
    k = p % 32
    lo = k < 16
    w[p[lo], g[lo]] = 2.0 ** k[lo]
    w[p[~lo], WPC + g[~lo]] = 2.0 ** (k[~lo] - 16)
    return jnp.asarray(w, jnp.bfloat16)


def _sum_weights():
    w = np.zeros((2 * WPC, 128), np.float32)
    j = np.arange(2 * WPC)
    w[j, (j % WPC) // 16] = 1.0
    return jnp.asarray(w, jnp.bfloat16)


def _mask_body(nx_ref, xyzt_ref, w_ref, ws_ref, out_ref, sum_ref):
    qx = nx_ref[:, 0:1]
    qy = nx_ref[:, 1:2]
    qz = nx_ref[:, 2:3]
    px = xyzt_ref[0:1, :]
    py = xyzt_ref[1:2, :]
    pz = xyzt_ref[2:3, :]
    dx = qx - px
    dy = qy - py
    dz = qz - pz
    d2 = dx * dx + dy * dy + dz * dz
    m = (d2 < R2).astype(jnp.bfloat16)
    pk = lax.dot_general(m, w_ref[...], (((1,), (0,)), ((), ())),
                         preferred_element_type=jnp.float32)
    lo = pk[:, :WPC].astype(jnp.int32)
    hi = pk[:, WPC:].astype(jnp.int32)
    out_ref[...] = lo | (hi << 16)

    pc = pl.program_id(1)
    gs = lax.dot_general(pk.astype(jnp.bfloat16), ws_ref[...],
                         (((1,), (0,)), ((), ())),
                         preferred_element_type=jnp.float32)
    g8 = lax.broadcasted_iota(jnp.int32, (QB, 8), 1) + 8 * pc
    bits = (gs[:, :8] > 0.0).astype(jnp.int32) << g8
    part = jnp.sum(bits, axis=1, keepdims=True)
    prev = jnp.where(pc == 0, 0, sum_ref[...])
    sum_ref[...] = prev + part


def _mask_bits(nxq, xyzt, w, ws):
    nq = nxq.shape[0]
    return pl.pallas_call(
        _mask_body,
        grid=(nq // QB, N // PCM),
        in_specs=[
            pl.BlockSpec((QB, 3), lambda qb, pc: (qb, 0)),
            pl.BlockSpec((3, PCM), lambda qb, pc: (0, pc)),
            pl.BlockSpec((PCM, 2 * WPC), lambda qb, pc: (0, 0)),
            pl.BlockSpec((2 * WPC, 128), lambda qb, pc: (0, 0)),
        ],
        out_specs=[
            pl.BlockSpec((QB, WPC), lambda qb, pc: (qb, pc)),
            pl.BlockSpec((QB, 1), lambda qb, pc: (qb, 0)),
        ],
        out_shape=[
            jax.ShapeDtypeStruct((nq, NWORDS), jnp.int32),
            jax.ShapeDtypeStruct((nq, 1), jnp.int32),
        ],
    )(nxq, xyzt, w, ws)


PC = 2048


def _table_body(f_ref, out_ref):
    out_ref[...] = f_ref[...].T


def _build_table(features):
    return pl.pallas_call(
        _table_body,
        grid=(N // PC,),
        in_specs=[
            pl.BlockSpec((CF, PC), lambda pc: (0, pc)),
        ],
        out_specs=pl.BlockSpec((PC, CF), lambda pc: (pc, 0)),
        out_shape=jax.ShapeDtypeStruct((N, CF), jnp.float32),
    )(features)


SC_NC = 2
SC_NS = 16
L = 16
NW = SC_NC * SC_NS
QPW = NQ // NW
QCH = 32
GQ = 4
NGRP = NWORDS // L


def _make_sc_body(nq):
    qpw = nq // NW

    def _sc_body(mask_hbm, summ_hbm, table_hbm, xyzf_hbm,
                 gath_hbm, idn_hbm, xyzg_hbm,
                 mask_v, sum_v, idx_v, idn_v, rows_v, rows_w, xyz_v, xyzg_v,
                 gsem0, gsem1, wsem0, wsem1):
        wid = lax.axis_index("s") * SC_NC + lax.axis_index("c")
        q0 = wid * qpw
        iota = lax.broadcasted_iota(jnp.int32, (L,), 0)

        pltpu.sync_copy(xyzf_hbm, xyz_v)

        @pl.loop(0, qpw // QCH)
        def _chunk(qc):
            pltpu.sync_copy(mask_hbm.at[pl.ds((q0 + qc * QCH) * NWORDS,
                                              QCH * NWORDS)], mask_v)
            pltpu.sync_copy(summ_hbm.at[pl.ds(q0 + qc * QCH, QCH)], sum_v)

            @pl.loop(0, QCH)
            def _query(ql):
                q = qc * QCH + ql
                sv = sum_v[pl.ds((ql // L) * L, L)]
                sq = jnp.sum(jnp.where(iota == ql % L, sv, 0))

                def extract_word(wv, base, cnt):
                    wb = jnp.broadcast_to(wv, (L,))

                    def half(h, cnt):
                        bits = lax.shift_right_logical(wb, iota + 16 * h) & 1
                        cs = jnp.cumsum(bits)
                        pos = cnt + cs - 1
                        sm = (bits > 0) & (pos < S)
                        posc = jnp.clip(pos, 0, S - 1)
                        vals = base + 16 * h + iota
                        plsc.store_scatter(idx_v, [q * S + posc], vals,
                                           mask=sm)
                        return cnt + jnp.sum(bits)

                    cnt = half(0, cnt)
                    cnt = half(1, cnt)
                    return cnt

                def group(g, cnt):
                    gbit = lax.shift_right_logical(sq, g) & 1

                    def extract(cnt):
                        words = mask_v[pl.ds(ql * NWORDS + g * L, L)]

                        def word(i, cnt):
                            wv = jnp.sum(jnp.where(iota == i, words, 0))
                            return lax.cond(
                                wv != 0,
                                lambda c: extract_word(
                                    wv, (g * L + i) * 32, c),
                                lambda c: c, cnt)
                        return lax.fori_loop(0, L, word, cnt)

                    return lax.cond((gbit > 0) & (cnt < S), extract,
                                    lambda c: c, cnt)

                cnt = lax.fori_loop(0, NGRP, group, jnp.int32(0))

                cntc = jnp.minimum(cnt, S)
                f16 = idx_v[pl.ds(q * S, L)]
                first = jnp.sum(jnp.where(iota == 0, f16, 0))
                first = jnp.where(cnt > 0, first, 0)

                @pl.loop(0, 2)
                def _fill(h):
                    sl = iota + 16 * h
                    cur = idx_v[pl.ds(q * S + 16 * h, L)]
                    idx_v[pl.ds(q * S + 16 * h, L)] = jnp.where(
                        sl < cntc, cur, first)
                    idn_v[pl.ds(q * S + 16 * h, L)] = (sl < cntc).astype(
                        jnp.int32)

        pltpu.sync_copy(idn_v, idn_hbm.at[pl.ds(q0 * S, qpw * S)])

        @pl.loop(0, (qpw * S) // L)
        def _xg(v):
            idxv = idx_v[pl.ds(v * L, L)]
            for c in range(3):
                g = plsc.load_gather(xyz_v, [idxv + c * N])
                xyzg_v[pl.ds(c * qpw * S + v * L, L)] = g

        @pl.loop(0, 3)
        def _xout(c):
            pltpu.sync_copy(
                xyzg_v.at[pl.ds(c * qpw * S, qpw * S)],
                xyzg_hbm.at[pl.ds(c * nq * S + q0 * S, qpw * S)])

        rows2 = (rows_v, rows_w)
        gsem = (gsem0, gsem1)
        wsem = (wsem0, wsem1)

        def g_copy(u, rb, sm):
            idxs = idx_v.at[pl.ds(u * GQ * S, GQ * S)]
            return pltpu.make_async_copy(table_hbm.at[idxs], rb, sm)

        def w_copy(u, rb, sm):
            dst = gath_hbm.at[pl.ds((q0 + u * GQ) * S, GQ * S)]
            return pltpu.make_async_copy(rb, dst, sm)

        ng = qpw // GQ

        @pl.loop(0, ng // 2)
        def _g2(t):
            for s in range(2):
                u = 2 * t + s
                o = 1 - s

                @pl.when(u >= 1)
                def _():
                    g_copy(u - 1, rows2[o], gsem[o]).wait()
                    w_copy(u - 1, rows2[o], wsem[o]).start()

                @pl.when(u >= 2)
                def _():
                    w_copy(u - 2, rows2[s], wsem[s]).wait()

                g_copy(u, rows2[s], gsem[s]).start()

        g_copy(ng - 1, rows2[1], gsem[1]).wait()
        w_copy(ng - 1, rows2[1], wsem[1]).start()
        w_copy(ng - 2, rows2[0], wsem[0]).wait()
        w_copy(ng - 1, rows2[1], wsem[1]).wait()

    return _sc_body


def _sc_select_gather(maskbits, summ, table, xyzf):
    nq = maskbits.shape[0]
    qpw = nq // NW
    mesh = plsc.VectorSubcoreMesh(core_axis_name="c", subcore_axis_name="s")
    cp = pltpu.CompilerParams()
    fields = pltpu.CompilerParams.__dataclass_fields__
    if "needs_layout_passes" in fields:
        cp = dataclasses.replace(cp, needs_layout_passes=False)
    if "use_tc_tiling_on_sc" in fields:
        cp = dataclasses.replace(cp, use_tc_tiling_on_sc=False)
    kern = pl.kernel(
        _make_sc_body(nq),
        mesh=mesh,
        compiler_params=cp,
        out_type=[
            jax.ShapeDtypeStruct((nq * S, CF), jnp.float32),
            jax.ShapeDtypeStruct((nq * S,), jnp.int32),
            jax.ShapeDtypeStruct((3 * nq * S,), jnp.float32),
        ],
        scratch_types=[
            pltpu.VMEM((QCH * NWORDS,), jnp.int32),
            pltpu.VMEM((QCH,), jnp.int32),
            pltpu.VMEM((qpw * S,), jnp.int32),
            pltpu.VMEM((qpw * S,), jnp.int32),
            pltpu.VMEM((GQ * S, CF), jnp.float32),
            pltpu.VMEM((GQ * S, CF), jnp.float32),
            pltpu.VMEM((3 * N,), jnp.float32),
            pltpu.VMEM((3 * qpw * S,), jnp.float32),
            pltpu.SemaphoreType.DMA,
            pltpu.SemaphoreType.DMA,
            pltpu.SemaphoreType.DMA,
            pltpu.SemaphoreType.DMA,
        ],
    )
    return kern(maskbits.reshape(nq * NWORDS), summ.reshape(nq),
                table, xyzf)


RB = 4096


def _final_body(rows_ref, xyzg_ref, ctr_ref, out_ref):
    t = rows_ref[...].T
    out_ref[0, :CF, :] = t
    xyzp = xyzg_ref[...] - ctr_ref[0]
    pad = jnp.zeros((CO2 - CO, RB), jnp.float32)
    out_ref[0, CF:, :] = jnp.concatenate([xyzp, pad], axis=0)


def _finalize(gath, xyzg, ctr):
    bh = ctr.shape[0]
    nb = (P * S) // RB
    return pl.pallas_call(
        _final_body,
        grid=(bh, nb),
        in_specs=[
            pl.BlockSpec((RB, CF), lambda b, pb: (b * nb + pb, 0)),
            pl.BlockSpec((3, RB), lambda b, pb: (0, b * nb + pb)),
            pl.BlockSpec((1, 3, RB), lambda b, pb: (b, 0, pb)),
        ],
        out_specs=pl.BlockSpec((1, CO2, RB), lambda b, pb: (b, 0, pb)),
        out_shape=jax.ShapeDtypeStruct((bh, CO2, P * S), jnp.float32),
    )(gath, xyzg, ctr)


def kernel(xyz, new_xyz, features):
    nxq = new_xyz.reshape(NQ, 3)
    xyzt = xyz.T
    w = _pack_weights()
    ws = _sum_weights()
    table = _build_table(features)
    xyzf = xyzt.reshape(3 * N)
    ctr = jnp.repeat(new_xyz.transpose(0, 2, 1), S, axis=2)

    nh = NQ // 2
    bh = B // 2
    os_, idns = [], []
    for h in range(2):
        mb, summ = _mask_bits(nxq[h * nh:(h + 1) * nh], xyzt, w, ws)
        gath, idn, xyzg = _sc_select_gather(mb, summ, table, xyzf)
        o = _finalize(gath, xyzg.reshape(3, nh * S),
                      ctr[h * bh:(h + 1) * bh])
        os_.append(o)
        idns.append(idn)

    o = jnp.concatenate(os_, axis=0)
    idn = jnp.concatenate(idns, axis=0).reshape(B, P, S)
    nf = jnp.concatenate([o[:, CF:CO], o[:, :CF]], axis=1)
    return nf.reshape(B, CO, P, S), idn


# speedup vs baseline: 1.0120x; 1.0120x over previous
import dataclasses
import functools

import numpy as np
import jax
import jax.numpy as jnp
from jax import lax
from jax.experimental import pallas as pl
from jax.experimental.pallas import tpu as pltpu
from jax.experimental.pallas import tpu_sc as plsc

N = 16384
B = 4
P = 1024
NQ = B * P
S = 32
CF = 64
CO = 3 + CF
CP = 128
CO2 = 72
R2 = 0.2 * 0.2

NWORDS = N // 32


QB = 512
PCM = 4096
WPC = PCM // 32


def _pack_weights():
    w = np.zeros((PCM, 2 * WPC), np.float32)
    p = np.arange(PCM)
    g = p // 32
    k = p % 32
    lo = k < 16
    w[p[lo], g[lo]] = 2.0 ** k[lo]
    w[p[~lo], WPC + g[~lo]] = 2.0 ** (k[~lo] - 16)
    return jnp.asarray(w, jnp.bfloat16)


def _sum_weights():
    w = np.zeros((2 * WPC, 128), np.float32)
    j = np.arange(2 * WPC)
    w[j, (j % WPC) // 16] = 1.0
    return jnp.asarray(w, jnp.bfloat16)


def _mask_body(nx_ref, xyzt_ref, w_ref, ws_ref, out_ref, sum_ref):
    qx = nx_ref[:, 0:1]
    qy = nx_ref[:, 1:2]
    qz = nx_ref[:, 2:3]
    px = xyzt_ref[0:1, :]
    py = xyzt_ref[1:2, :]
    pz = xyzt_ref[2:3, :]
    dx = qx - px
    dy = qy - py
    dz = qz - pz
    d2 = dx * dx + dy * dy + dz * dz
    m = (d2 < R2).astype(jnp.bfloat16)
    pk = lax.dot_general(m, w_ref[...], (((1,), (0,)), ((), ())),
                         preferred_element_type=jnp.float32)
    lo = pk[:, :WPC].astype(jnp.int32)
    hi = pk[:, WPC:].astype(jnp.int32)
    out_ref[...] = lo | (hi << 16)

    pc = pl.program_id(1)
    gs = lax.dot_general(pk.astype(jnp.bfloat16), ws_ref[...],
                         (((1,), (0,)), ((), ())),
                         preferred_element_type=jnp.float32)
    g8 = lax.broadcasted_iota(jnp.int32, (QB, 8), 1) + 8 * pc
    bits = (gs[:, :8] > 0.0).astype(jnp.int32) << g8
    part = jnp.sum(bits, axis=1, keepdims=True)
    prev = jnp.where(pc == 0, 0, sum_ref[...])
    sum_ref[...] = prev + part


def _mask_bits(nxq, xyzt, w, ws):
    nq = nxq.shape[0]
    return pl.pallas_call(
        _mask_body,
        grid=(nq // QB, N // PCM),
        in_specs=[
            pl.BlockSpec((QB, 3), lambda qb, pc: (qb, 0)),
            pl.BlockSpec((3, PCM), lambda qb, pc: (0, pc)),
            pl.BlockSpec((PCM, 2 * WPC), lambda qb, pc: (0, 0)),
            pl.BlockSpec((2 * WPC, 128), lambda qb, pc: (0, 0)),
        ],
        out_specs=[
            pl.BlockSpec((QB, WPC), lambda qb, pc: (qb, pc)),
            pl.BlockSpec((QB, 1), lambda qb, pc: (qb, 0)),
        ],
        out_shape=[
            jax.ShapeDtypeStruct((nq, NWORDS), jnp.int32),
            jax.ShapeDtypeStruct((nq, 1), jnp.int32),
        ],
    )(nxq, xyzt, w, ws)


PC = 2048


def _table_body(f_ref, out_ref):
    out_ref[...] = f_ref[...].T


def _build_table(features):
    return pl.pallas_call(
        _table_body,
        grid=(N // PC,),
        in_specs=[
            pl.BlockSpec((CF, PC), lambda pc: (0, pc)),
        ],
        out_specs=pl.BlockSpec((PC, CF), lambda pc: (pc, 0)),
        out_shape=jax.ShapeDtypeStruct((N, CF), jnp.float32),
    )(features)


SC_NC = 2
SC_NS = 16
L = 16
NW = SC_NC * SC_NS
QPW = NQ // NW
QCH = 32
GQ = 4
NGRP = NWORDS // L


def _make_sc_body(nq):
    qpw = nq // NW

    def _sc_body(mask_hbm, summ_hbm, table_hbm, xyzf_hbm,
                 gath_hbm, idn_hbm, xyzg_hbm,
                 mask_v, sum_v, idx_v, idn_v, rows_v, xyz_v, xyzg_v, gsem0):
        wid = lax.axis_index("s") * SC_NC + lax.axis_index("c")
        q0 = wid * qpw
        iota = lax.broadcasted_iota(jnp.int32, (L,), 0)

        pltpu.sync_copy(xyzf_hbm, xyz_v)

        @pl.loop(0, qpw // QCH)
        def _chunk(qc):
            pltpu.sync_copy(mask_hbm.at[pl.ds((q0 + qc * QCH) * NWORDS,
                                              QCH * NWORDS)], mask_v)
            pltpu.sync_copy(summ_hbm.at[pl.ds(q0 + qc * QCH, QCH)], sum_v)

            @pl.loop(0, QCH)
            def _query(ql):
                q = qc * QCH + ql
                sv = sum_v[pl.ds((ql // L) * L, L)]
                sq = jnp.sum(jnp.where(iota == ql % L, sv, 0))

                def extract_word(wv, base, cnt):
                    wb = jnp.broadcast_to(wv, (L,))

                    def half(h, cnt):
                        bits = lax.shift_right_logical(wb, iota + 16 * h) & 1
                        cs = jnp.cumsum(bits)
                        pos = cnt + cs - 1
                        sm = (bits > 0) & (pos < S)
                        posc = jnp.clip(pos, 0, S - 1)
                        vals = base + 16 * h + iota
                        plsc.store_scatter(idx_v, [q * S + posc], vals,
                                           mask=sm)
                        return cnt + jnp.sum(bits)

                    cnt = half(0, cnt)
                    cnt = half(1, cnt)
                    return cnt

                def group(g, cnt):
                    gbit = lax.shift_right_logical(sq, g) & 1

                    def extract(cnt):
                        words = mask_v[pl.ds(ql * NWORDS + g * L, L)]

                        def word(i, cnt):
                            wv = jnp.sum(jnp.where(iota == i, words, 0))
                            return lax.cond(
                                wv != 0,
                                lambda c: extract_word(
                                    wv, (g * L + i) * 32, c),
                                lambda c: c, cnt)
                        return lax.fori_loop(0, L, word, cnt)

                    return lax.cond((gbit > 0) & (cnt < S), extract,
                                    lambda c: c, cnt)

                cnt = lax.fori_loop(0, NGRP, group, jnp.int32(0))

                cntc = jnp.minimum(cnt, S)
                f16 = idx_v[pl.ds(q * S, L)]
                first = jnp.sum(jnp.where(iota == 0, f16, 0))
                first = jnp.where(cnt > 0, first, 0)

                @pl.loop(0, 2)
                def _fill(h):
                    sl = iota + 16 * h
                    cur = idx_v[pl.ds(q * S + 16 * h, L)]
                    idx_v[pl.ds(q * S + 16 * h, L)] = jnp.where(
                        sl < cntc, cur, first)
                    idn_v[pl.ds(q * S + 16 * h, L)] = (sl < cntc).astype(
                        jnp.int32)

        pltpu.sync_copy(idn_v, idn_hbm.at[pl.ds(q0 * S, qpw * S)])

        @pl.loop(0, (qpw * S) // L)
        def _xg(v):
            idxv = idx_v[pl.ds(v * L, L)]
            for c in range(3):
                g = plsc.load_gather(xyz_v, [idxv + c * N])
                xyzg_v[pl.ds(c * qpw * S + v * L, L)] = g

        @pl.loop(0, 3)
        def _xout(c):
            pltpu.sync_copy(
                xyzg_v.at[pl.ds(c * qpw * S, qpw * S)],
                xyzg_hbm.at[pl.ds(c * nq * S + q0 * S, qpw * S)])

        @pl.loop(0, qpw // GQ)
        def _gather(gc):
            idxs = idx_v.at[pl.ds(gc * GQ * S, GQ * S)]
            pltpu.async_copy(table_hbm.at[idxs], rows_v, gsem0).wait()
            pltpu.sync_copy(rows_v,
                            gath_hbm.at[pl.ds((q0 + gc * GQ) * S, GQ * S)])

    return _sc_body


def _sc_select_gather(maskbits, summ, table, xyzf):
    nq = maskbits.shape[0]
    qpw = nq // NW
    mesh = plsc.VectorSubcoreMesh(core_axis_name="c", subcore_axis_name="s")
    cp = pltpu.CompilerParams()
    fields = pltpu.CompilerParams.__dataclass_fields__
    if "needs_layout_passes" in fields:
        cp = dataclasses.replace(cp, needs_layout_passes=False)
    if "use_tc_tiling_on_sc" in fields:
        cp = dataclasses.replace(cp, use_tc_tiling_on_sc=False)
    kern = pl.kernel(
        _make_sc_body(nq),
        mesh=mesh,
        compiler_params=cp,
        out_type=[
            jax.ShapeDtypeStruct((nq * S, CF), jnp.float32),
            jax.ShapeDtypeStruct((nq * S,), jnp.int32),
            jax.ShapeDtypeStruct((3 * nq * S,), jnp.float32),
        ],
        scratch_types=[
            pltpu.VMEM((QCH * NWORDS,), jnp.int32),
            pltpu.VMEM((QCH,), jnp.int32),
            pltpu.VMEM((qpw * S,), jnp.int32),
            pltpu.VMEM((qpw * S,), jnp.int32),
            pltpu.VMEM((GQ * S, CF), jnp.float32),
            pltpu.VMEM((3 * N,), jnp.float32),
            pltpu.VMEM((3 * qpw * S,), jnp.float32),
            pltpu.SemaphoreType.DMA,
        ],
    )
    return kern(maskbits.reshape(nq * NWORDS), summ.reshape(nq),
                table, xyzf)


RB = 4096


def _final_body(rows_ref, xyzg_ref, ctr_ref, out_ref):
    t = rows_ref[...].T
    out_ref[0, :CF, :] = t
    xyzp = xyzg_ref[...] - ctr_ref[0]
    pad = jnp.zeros((CO2 - CO, RB), jnp.float32)
    out_ref[0, CF:, :] = jnp.concatenate([xyzp, pad], axis=0)


def _finalize(gath, xyzg, ctr):
    bh = ctr.shape[0]
    nb = (P * S) // RB
    return pl.pallas_call(
        _final_body,
        grid=(bh, nb),
        in_specs=[
            pl.BlockSpec((RB, CF), lambda b, pb: (b * nb + pb, 0)),
            pl.BlockSpec((3, RB), lambda b, pb: (0, b * nb + pb)),
            pl.BlockSpec((1, 3, RB), lambda b, pb: (b, 0, pb)),
        ],
        out_specs=pl.BlockSpec((1, CO2, RB), lambda b, pb: (b, 0, pb)),
        out_shape=jax.ShapeDtypeStruct((bh, CO2, P * S), jnp.float32),
    )(gath, xyzg, ctr)


def kernel(xyz, new_xyz, features):
    nxq = new_xyz.reshape(NQ, 3)
    xyzt = xyz.T
    w = _pack_weights()
    ws = _sum_weights()
    table = _build_table(features)
    xyzf = xyzt.reshape(3 * N)
    ctr = jnp.repeat(new_xyz.transpose(0, 2, 1), S, axis=2)

    nsl = 4
    nh = NQ // nsl
    bh = B // nsl
    os_, idns = [], []
    for h in range(nsl):
        mb, summ = _mask_bits(nxq[h * nh:(h + 1) * nh], xyzt, w, ws)
        gath, idn, xyzg = _sc_select_gather(mb, summ, table, xyzf)
        o = _finalize(gath, xyzg.reshape(3, nh * S),
                      ctr[h * bh:(h + 1) * bh])
        os_.append(o)
        idns.append(idn)

    o = jnp.concatenate(os_, axis=0)
    idn = jnp.concatenate(idns, axis=0).reshape(B, P, S)
    nf = jnp.concatenate([o[:, CF:CO], o[:, :CF]], axis=1)
    return nf.reshape(B, CO, P, S), idn
